# baseline (speedup 1.0000x reference)
import functools

import jax
import jax.numpy as jnp
from jax import lax
from jax.experimental import pallas as pl
from jax.experimental.pallas import tpu as pltpu
from jax.experimental.pallas import tpu_sc as plsc

E = 8
DIN = 768
DFF = 3072
DOUT = 768
S = 2048
TM = 128
EP = 128
P = S + E * TM
NT = P // TM
NCHUNK = S // TM


def _route_body(x_ref, wg_ref, bg_ref, dest_ref, src_ref, cnt_ref, oh_ref):
    logits = jnp.dot(x_ref[...], wg_ref[...],
                     preferred_element_type=jnp.float32) + bg_ref[...]
    lane = lax.broadcasted_iota(jnp.int32, (S, EP), 1)
    rowmax = jnp.max(logits, axis=1, keepdims=True)
    ismax = logits >= rowmax
    sel = jnp.min(jnp.where(ismax, lane, EP), axis=1, keepdims=True)
    oh = (lane == sel).astype(jnp.float32)
    oh_ref[...] = oh
    counts = jnp.sum(oh, axis=0, keepdims=True)
    padded = jnp.ceil(counts / TM) * TM
    r2 = lax.broadcasted_iota(jnp.int32, (EP, EP), 0)
    c2 = lax.broadcasted_iota(jnp.int32, (EP, EP), 1)
    starts = jnp.dot(padded, (r2 < c2).astype(jnp.float32),
                     preferred_element_type=jnp.float32)
    ltri = (r2 > c2).astype(jnp.float32)

    def body(i, running):
        oh_c = oh_ref[pl.ds(i * TM, TM), :]
        rank_c = jnp.dot(ltri, oh_c, preferred_element_type=jnp.float32) + running
        dest_c = jnp.sum(oh_c * (rank_c + starts), axis=1, keepdims=True)
        dest_ref[pl.ds(i * TM, TM), :] = dest_c.astype(jnp.int32)
        return running + jnp.sum(oh_c, axis=0, keepdims=True)

    lax.fori_loop(0, NCHUNK, body, jnp.zeros((1, EP), jnp.float32))

    destf = dest_ref[...].astype(jnp.float32)
    tokf = lax.broadcasted_iota(jnp.int32, (S, 1), 0).astype(jnp.float32)
    lanef = lax.broadcasted_iota(jnp.int32, (1, EP), 1).astype(jnp.float32)

    def body2(i, carry):
        pv = lanef + (i * TM).astype(jnp.float32)
        dmask = (destf == pv).astype(jnp.float32)
        src_c = jnp.sum(dmask * tokf, axis=0, keepdims=True)
        src_ref[:, pl.ds(i * TM, TM)] = src_c.astype(jnp.int32)
        return carry

    lax.fori_loop(0, NT, body2, 0)
    cnt_ref[...] = counts.astype(jnp.int32)


def _route(x2d, wgp, bgp):
    return pl.pallas_call(
        _route_body,
        out_shape=(
            jax.ShapeDtypeStruct((S, 1), jnp.int32),
            jax.ShapeDtypeStruct((1, P), jnp.int32),
            jax.ShapeDtypeStruct((1, EP), jnp.int32),
        ),
        scratch_shapes=[pltpu.VMEM((S, EP), jnp.float32)],
    )(x2d, wgp, bgp)


def _ffn_body(eot_ref, xs_ref, w1_ref, b1_ref, w2_ref, b2_ref, out_ref):
    del eot_ref
    a = xs_ref[...].astype(jnp.bfloat16)
    h = jnp.dot(a, w1_ref[0], preferred_element_type=jnp.float32) + b1_ref[0]
    g = 0.5 * h * (1.0 + lax.erf(h * (2.0 ** -0.5)))
    y = jnp.dot(g.astype(jnp.bfloat16), w2_ref[0],
                preferred_element_type=jnp.float32) + b2_ref[0]
    out_ref[...] = y


def _ffn(eot, xs, w1, b1, w2, b2):
    grid_spec = pltpu.PrefetchScalarGridSpec(
        num_scalar_prefetch=1,
        grid=(NT,),
        in_specs=[
            pl.BlockSpec((TM, DIN), lambda t, eot: (t, 0)),
            pl.BlockSpec((1, DIN, DFF), lambda t, eot: (eot[t], 0, 0)),
            pl.BlockSpec((1, 1, DFF), lambda t, eot: (eot[t], 0, 0)),
            pl.BlockSpec((1, DFF, DOUT), lambda t, eot: (eot[t], 0, 0)),
            pl.BlockSpec((1, 1, DOUT), lambda t, eot: (eot[t], 0, 0)),
        ],
        out_specs=pl.BlockSpec((TM, DOUT), lambda t, eot: (t, 0)),
    )
    return pl.pallas_call(
        _ffn_body,
        grid_spec=grid_spec,
        out_shape=jax.ShapeDtypeStruct((P, DOUT), jnp.float32),
    )(eot, xs, w1.astype(jnp.bfloat16), b1.reshape(E, 1, DFF),
      w2.astype(jnp.bfloat16), b2.reshape(E, 1, DOUT))


def _sc_gather(table, idx, nrows, d):
    info = plsc.get_sparse_core_info()
    nw = info.num_cores * info.num_subcores
    bpw = nrows // nw
    mesh = plsc.VectorSubcoreMesh(core_axis_name="c", subcore_axis_name="s")

    @functools.partial(
        pl.kernel,
        mesh=mesh,
        out_type=jax.ShapeDtypeStruct((nrows, d), jnp.float32),
        scratch_types=[
            pltpu.VMEM((bpw,), jnp.int32),
            pltpu.VMEM((bpw, d), jnp.float32),
            pltpu.SemaphoreType.DMA,
        ],
    )
    def k(table_hbm, idx_hbm, out_hbm, idx_v, rows_v, sem):
        wid = lax.axis_index("s") * info.num_cores + lax.axis_index("c")
        base = wid * bpw
        pltpu.sync_copy(idx_hbm.at[pl.ds(base, bpw)], idx_v)
        pltpu.async_copy(table_hbm.at[idx_v], rows_v, sem).wait()
        pltpu.sync_copy(rows_v, out_hbm.at[pl.ds(base, bpw)])

    return k(table, idx)


def kernel(x, Wg, bg, W1, b1, W2, b2):
    x2d = x.reshape(S, DIN)
    wgp = jnp.pad(Wg, ((0, 0), (0, EP - E)))
    bgp = jnp.pad(bg, (0, EP - E), constant_values=-1e30).reshape(1, EP)

    dest, src, cnt = _route(x2d, wgp, bgp)

    c8 = cnt[0, :E]
    pc = ((c8 + TM - 1) // TM) * TM
    ends = jnp.cumsum(pc)
    tile_start = jnp.arange(NT, dtype=jnp.int32) * TM
    eot = jnp.clip(jnp.searchsorted(ends, tile_start, side="right"),
                   0, E - 1).astype(jnp.int32)

    xs = _sc_gather(x2d, src.reshape(P), P, DIN)
    ys = _ffn(eot, xs, W1, b1, W2, b2)
    out2d = _sc_gather(ys, dest.reshape(S), S, DOUT)
    return out2d.reshape(1, S, DOUT)


# speedup vs baseline: 1.2363x; 1.2363x over previous
import functools

import jax
import jax.numpy as jnp
from jax import lax
from jax.experimental import pallas as pl
from jax.experimental.pallas import tpu as pltpu
from jax.experimental.pallas import tpu_sc as plsc

E = 8
DIN = 768
DFF = 3072
DOUT = 768
S = 2048
TM = 128
EP = 128
P = S + E * TM
NT = P // TM
NCHUNK = S // TM


def _route_body(x_ref, wg_ref, bg_ref, dest_ref, src_ref, cnt_ref, oh_ref):
    logits = jnp.dot(x_ref[...], wg_ref[...],
                     preferred_element_type=jnp.float32) + bg_ref[...]
    lane = lax.broadcasted_iota(jnp.int32, (S, EP), 1)
    rowmax = jnp.max(logits, axis=1, keepdims=True)
    ismax = logits >= rowmax
    sel = jnp.min(jnp.where(ismax, lane, EP), axis=1, keepdims=True)
    oh = (lane == sel).astype(jnp.float32)
    oh_ref[...] = oh
    counts = jnp.sum(oh, axis=0, keepdims=True)
    padded = jnp.ceil(counts / TM) * TM
    r2 = lax.broadcasted_iota(jnp.int32, (EP, EP), 0)
    c2 = lax.broadcasted_iota(jnp.int32, (EP, EP), 1)
    starts = jnp.dot(padded, (r2 < c2).astype(jnp.float32),
                     preferred_element_type=jnp.float32)
    ltri = (r2 > c2).astype(jnp.float32)

    def body(i, running):
        oh_c = oh_ref[pl.ds(i * TM, TM), :]
        rank_c = jnp.dot(ltri, oh_c, preferred_element_type=jnp.float32) + running
        dest_c = jnp.sum(oh_c * (rank_c + starts), axis=1, keepdims=True)
        dest_ref[pl.ds(i * TM, TM), :] = dest_c.astype(jnp.int32)
        return running + jnp.sum(oh_c, axis=0, keepdims=True)

    lax.fori_loop(0, NCHUNK, body, jnp.zeros((1, EP), jnp.float32))

    destf = dest_ref[...].astype(jnp.float32)
    tokf = lax.broadcasted_iota(jnp.int32, (S, 1), 0).astype(jnp.float32)
    lanef = lax.broadcasted_iota(jnp.int32, (1, EP), 1).astype(jnp.float32)

    def body2(i, carry):
        pv = lanef + (i * TM).astype(jnp.float32)
        dmask = (destf == pv).astype(jnp.float32)
        src_c = jnp.sum(dmask * tokf, axis=0, keepdims=True)
        src_ref[:, pl.ds(i * TM, TM)] = src_c.astype(jnp.int32)
        return carry

    lax.fori_loop(0, NT, body2, 0)
    cnt_ref[...] = counts.astype(jnp.int32)


def _route(x2d, wgp, bgp):
    return pl.pallas_call(
        _route_body,
        out_shape=(
            jax.ShapeDtypeStruct((S, 1), jnp.int32),
            jax.ShapeDtypeStruct((1, P), jnp.int32),
            jax.ShapeDtypeStruct((1, EP), jnp.int32),
        ),
        scratch_shapes=[pltpu.VMEM((S, EP), jnp.float32)],
    )(x2d, wgp, bgp)


def _ffn_body(eot_ref, xs_ref, w1_ref, b1_ref, w2_ref, b2_ref, out_ref):
    del eot_ref
    a = xs_ref[...].astype(jnp.bfloat16)
    h = jnp.dot(a, w1_ref[0].astype(jnp.bfloat16),
                preferred_element_type=jnp.float32) + b1_ref[0]
    g = 0.5 * h * (1.0 + lax.erf(h * (2.0 ** -0.5)))
    y = jnp.dot(g.astype(jnp.bfloat16), w2_ref[0].astype(jnp.bfloat16),
                preferred_element_type=jnp.float32) + b2_ref[0]
    out_ref[...] = y


def _ffn(eot, xs, w1, b1, w2, b2):
    grid_spec = pltpu.PrefetchScalarGridSpec(
        num_scalar_prefetch=1,
        grid=(NT,),
        in_specs=[
            pl.BlockSpec((TM, DIN), lambda t, eot: (t, 0)),
            pl.BlockSpec((1, DIN, DFF), lambda t, eot: (eot[t], 0, 0)),
            pl.BlockSpec((1, 1, DFF), lambda t, eot: (eot[t], 0, 0)),
            pl.BlockSpec((1, DFF, DOUT), lambda t, eot: (eot[t], 0, 0)),
            pl.BlockSpec((1, 1, DOUT), lambda t, eot: (eot[t], 0, 0)),
        ],
        out_specs=pl.BlockSpec((TM, DOUT), lambda t, eot: (t, 0)),
    )
    return pl.pallas_call(
        _ffn_body,
        grid_spec=grid_spec,
        out_shape=jax.ShapeDtypeStruct((P, DOUT), jnp.float32),
    )(eot, xs, w1, b1.reshape(E, 1, DFF), w2, b2.reshape(E, 1, DOUT))


def _sc_gather(table, idx, nrows, d):
    info = plsc.get_sparse_core_info()
    nw = info.num_cores * info.num_subcores
    bpw = nrows // nw
    mesh = plsc.VectorSubcoreMesh(core_axis_name="c", subcore_axis_name="s")

    @functools.partial(
        pl.kernel,
        mesh=mesh,
        out_type=jax.ShapeDtypeStruct((nrows, d), jnp.float32),
        scratch_types=[
            pltpu.VMEM((bpw,), jnp.int32),
            pltpu.VMEM((bpw, d), jnp.float32),
            pltpu.SemaphoreType.DMA,
        ],
    )
    def k(table_hbm, idx_hbm, out_hbm, idx_v, rows_v, sem):
        wid = lax.axis_index("s") * info.num_cores + lax.axis_index("c")
        base = wid * bpw
        pltpu.sync_copy(idx_hbm.at[pl.ds(base, bpw)], idx_v)
        pltpu.async_copy(table_hbm.at[idx_v], rows_v, sem).wait()
        pltpu.sync_copy(rows_v, out_hbm.at[pl.ds(base, bpw)])

    return k(table, idx)


def kernel(x, Wg, bg, W1, b1, W2, b2):
    x2d = x.reshape(S, DIN)
    wgp = jnp.pad(Wg, ((0, 0), (0, EP - E)))
    bgp = jnp.pad(bg, (0, EP - E), constant_values=-1e30).reshape(1, EP)

    dest, src, cnt = _route(x2d, wgp, bgp)

    c8 = cnt[0, :E]
    pc = ((c8 + TM - 1) // TM) * TM
    ends = jnp.cumsum(pc)
    tile_start = jnp.arange(NT, dtype=jnp.int32) * TM
    eot = jnp.clip(jnp.searchsorted(ends, tile_start, side="right"),
                   0, E - 1).astype(jnp.int32)

    xs = _sc_gather(x2d, src.reshape(P), P, DIN)
    ys = _ffn(eot, xs, W1, b1, W2, b2)
    out2d = _sc_gather(ys, dest.reshape(S), S, DOUT)
    return out2d.reshape(1, S, DOUT)


# speedup vs baseline: 1.6655x; 1.3471x over previous
import functools

import jax
import jax.numpy as jnp
from jax import lax
from jax.experimental import pallas as pl
from jax.experimental.pallas import tpu as pltpu
from jax.experimental.pallas import tpu_sc as plsc

E = 8
DIN = 768
DFF = 3072
DOUT = 768
S = 2048
TM = 128
EP = 128
P = S + E * TM
NT = P // TM
NCHUNK = S // TM


def _route_body(x_ref, wg_ref, bg_ref, dest_ref, src_ref, cnt_ref, oh_ref):
    logits = jnp.dot(x_ref[...], wg_ref[...],
                     preferred_element_type=jnp.float32) + bg_ref[...]
    lane = lax.broadcasted_iota(jnp.int32, (S, EP), 1)
    rowmax = jnp.max(logits, axis=1, keepdims=True)
    ismax = logits >= rowmax
    sel = jnp.min(jnp.where(ismax, lane, EP), axis=1, keepdims=True)
    oh = (lane == sel).astype(jnp.float32)
    oh_ref[...] = oh
    counts = jnp.sum(oh, axis=0, keepdims=True)
    padded = jnp.ceil(counts / TM) * TM
    r2 = lax.broadcasted_iota(jnp.int32, (EP, EP), 0)
    c2 = lax.broadcasted_iota(jnp.int32, (EP, EP), 1)
    starts = jnp.dot(padded, (r2 < c2).astype(jnp.float32),
                     preferred_element_type=jnp.float32)
    ltri = (r2 > c2).astype(jnp.float32)

    def body(i, running):
        oh_c = oh_ref[pl.ds(i * TM, TM), :]
        rank_c = jnp.dot(ltri, oh_c, preferred_element_type=jnp.float32) + running
        dest_c = jnp.sum(oh_c * (rank_c + starts), axis=1, keepdims=True)
        dest_ref[pl.ds(i * TM, TM), :] = dest_c.astype(jnp.int32)
        return running + jnp.sum(oh_c, axis=0, keepdims=True)

    lax.fori_loop(0, NCHUNK, body, jnp.zeros((1, EP), jnp.float32))

    destf = dest_ref[...].astype(jnp.float32)
    tokf = lax.broadcasted_iota(jnp.int32, (S, 1), 0).astype(jnp.float32)
    lanef = lax.broadcasted_iota(jnp.int32, (1, EP), 1).astype(jnp.float32)

    def body2(i, carry):
        pv = lanef + (i * TM).astype(jnp.float32)
        dmask = (destf == pv).astype(jnp.float32)
        hit = jnp.sum(dmask, axis=0, keepdims=True)
        src_c = jnp.sum(dmask * tokf, axis=0, keepdims=True)
        filler = pv - jnp.floor(pv / S) * S
        src_c = src_c + (1.0 - hit) * filler
        src_ref[:, pl.ds(i * TM, TM)] = src_c.astype(jnp.int32)
        return carry

    lax.fori_loop(0, NT, body2, 0)
    cnt_ref[...] = counts.astype(jnp.int32)


def _route(x2d, wgp, bgp):
    return pl.pallas_call(
        _route_body,
        out_shape=(
            jax.ShapeDtypeStruct((S, 1), jnp.int32),
            jax.ShapeDtypeStruct((1, P), jnp.int32),
            jax.ShapeDtypeStruct((1, EP), jnp.int32),
        ),
        scratch_shapes=[pltpu.VMEM((S, EP), jnp.float32)],
    )(x2d, wgp, bgp)


def _ffn_body(eot_ref, xs_ref, w1_ref, b1_ref, w2_ref, b2_ref, out_ref):
    del eot_ref
    a = xs_ref[...].astype(jnp.bfloat16)
    h = jnp.dot(a, w1_ref[0].astype(jnp.bfloat16),
                preferred_element_type=jnp.float32) + b1_ref[0]
    g = 0.5 * h * (1.0 + lax.erf(h * (2.0 ** -0.5)))
    y = jnp.dot(g.astype(jnp.bfloat16), w2_ref[0].astype(jnp.bfloat16),
                preferred_element_type=jnp.float32) + b2_ref[0]
    out_ref[...] = y


def _ffn(eot, xs, w1, b1, w2, b2):
    grid_spec = pltpu.PrefetchScalarGridSpec(
        num_scalar_prefetch=1,
        grid=(NT,),
        in_specs=[
            pl.BlockSpec((TM, DIN), lambda t, eot: (t, 0)),
            pl.BlockSpec((1, DIN, DFF), lambda t, eot: (eot[t], 0, 0)),
            pl.BlockSpec((1, 1, DFF), lambda t, eot: (eot[t], 0, 0)),
            pl.BlockSpec((1, DFF, DOUT), lambda t, eot: (eot[t], 0, 0)),
            pl.BlockSpec((1, 1, DOUT), lambda t, eot: (eot[t], 0, 0)),
        ],
        out_specs=pl.BlockSpec((TM, DOUT), lambda t, eot: (t, 0)),
    )
    return pl.pallas_call(
        _ffn_body,
        grid_spec=grid_spec,
        out_shape=jax.ShapeDtypeStruct((P, DOUT), jnp.float32),
    )(eot, xs, w1, b1.reshape(E, 1, DFF), w2, b2.reshape(E, 1, DOUT))


def _sc_gather(table, idx, nrows, d):
    info = plsc.get_sparse_core_info()
    nw = info.num_cores * info.num_subcores
    bpw = nrows // nw
    mesh = plsc.VectorSubcoreMesh(core_axis_name="c", subcore_axis_name="s")

    @functools.partial(
        pl.kernel,
        mesh=mesh,
        out_type=jax.ShapeDtypeStruct((nrows, d), jnp.float32),
        scratch_types=[
            pltpu.VMEM((bpw,), jnp.int32),
            pltpu.VMEM((bpw, d), jnp.float32),
            pltpu.SemaphoreType.DMA,
        ],
    )
    def k(table_hbm, idx_hbm, out_hbm, idx_v, rows_v, sem):
        wid = lax.axis_index("s") * info.num_cores + lax.axis_index("c")
        base = wid * bpw
        pltpu.sync_copy(idx_hbm.at[pl.ds(base, bpw)], idx_v)
        pltpu.async_copy(table_hbm.at[idx_v], rows_v, sem).wait()
        pltpu.sync_copy(rows_v, out_hbm.at[pl.ds(base, bpw)])

    return k(table, idx)


def kernel(x, Wg, bg, W1, b1, W2, b2):
    x2d = x.reshape(S, DIN)
    wgp = jnp.pad(Wg, ((0, 0), (0, EP - E)))
    bgp = jnp.pad(bg, (0, EP - E), constant_values=-1e30).reshape(1, EP)

    dest, src, cnt = _route(x2d, wgp, bgp)

    c8 = cnt[0, :E]
    pc = ((c8 + TM - 1) // TM) * TM
    ends = jnp.cumsum(pc)
    tile_start = jnp.arange(NT, dtype=jnp.int32) * TM
    eot = jnp.clip(jnp.searchsorted(ends, tile_start, side="right"),
                   0, E - 1).astype(jnp.int32)

    xs = _sc_gather(x2d, src.reshape(P), P, DIN)
    ys = _ffn(eot, xs, W1, b1, W2, b2)
    out2d = _sc_gather(ys, dest.reshape(S), S, DOUT)
    return out2d.reshape(1, S, DOUT)


# speedup vs baseline: 1.7665x; 1.0607x over previous
import functools

import jax
import jax.numpy as jnp
from jax import lax
from jax.experimental import pallas as pl
from jax.experimental.pallas import tpu as pltpu
from jax.experimental.pallas import tpu_sc as plsc

E = 8
DIN = 768
DFF = 3072
DOUT = 768
S = 2048
TM = 128
EP = 128
P = S + E * TM
NT = P // TM
NCHUNK = S // TM


def _route_body(x_ref, wg_ref, bg_ref, dest_ref, cnt_ref, oh_ref):
    logits = jnp.dot(x_ref[...], wg_ref[...],
                     preferred_element_type=jnp.float32) + bg_ref[...]
    lane = lax.broadcasted_iota(jnp.int32, (S, EP), 1)
    rowmax = jnp.max(logits, axis=1, keepdims=True)
    ismax = logits >= rowmax
    sel = jnp.min(jnp.where(ismax, lane, EP), axis=1, keepdims=True)
    oh = (lane == sel).astype(jnp.float32)
    oh_ref[...] = oh
    counts = jnp.sum(oh, axis=0, keepdims=True)
    padded = jnp.ceil(counts / TM) * TM
    r2 = lax.broadcasted_iota(jnp.int32, (EP, EP), 0)
    c2 = lax.broadcasted_iota(jnp.int32, (EP, EP), 1)
    starts = jnp.dot(padded, (r2 < c2).astype(jnp.float32),
                     preferred_element_type=jnp.float32)
    ltri = (r2 > c2).astype(jnp.float32)

    def body(i, running):
        oh_c = oh_ref[pl.ds(i * TM, TM), :]
        rank_c = jnp.dot(ltri, oh_c, preferred_element_type=jnp.float32) + running
        dest_c = jnp.sum(oh_c * (rank_c + starts), axis=1, keepdims=True)
        dest_ref[pl.ds(i * TM, TM), :] = dest_c.astype(jnp.int32)
        return running + jnp.sum(oh_c, axis=0, keepdims=True)

    lax.fori_loop(0, NCHUNK, body, jnp.zeros((1, EP), jnp.float32))
    cnt_ref[...] = counts.astype(jnp.int32)


def _route(x2d, wgp, bgp):
    return pl.pallas_call(
        _route_body,
        out_shape=(
            jax.ShapeDtypeStruct((S, 1), jnp.int32),
            jax.ShapeDtypeStruct((1, EP), jnp.int32),
        ),
        scratch_shapes=[pltpu.VMEM((S, EP), jnp.float32)],
    )(x2d, wgp, bgp)


def _ffn_body(eot_ref, xs_ref, w1_ref, b1_ref, w2_ref, b2_ref, out_ref):
    del eot_ref
    a = xs_ref[...].astype(jnp.bfloat16)
    h = jnp.dot(a, w1_ref[0].astype(jnp.bfloat16),
                preferred_element_type=jnp.float32) + b1_ref[0]
    g = 0.5 * h * (1.0 + lax.erf(h * (2.0 ** -0.5)))
    y = jnp.dot(g.astype(jnp.bfloat16), w2_ref[0].astype(jnp.bfloat16),
                preferred_element_type=jnp.float32) + b2_ref[0]
    out_ref[...] = y


def _ffn(eot, xs, w1, b1, w2, b2):
    grid_spec = pltpu.PrefetchScalarGridSpec(
        num_scalar_prefetch=1,
        grid=(NT,),
        in_specs=[
            pl.BlockSpec((TM, DIN), lambda t, eot: (t, 0)),
            pl.BlockSpec((1, DIN, DFF), lambda t, eot: (eot[t], 0, 0)),
            pl.BlockSpec((1, 1, DFF), lambda t, eot: (eot[t], 0, 0)),
            pl.BlockSpec((1, DFF, DOUT), lambda t, eot: (eot[t], 0, 0)),
            pl.BlockSpec((1, 1, DOUT), lambda t, eot: (eot[t], 0, 0)),
        ],
        out_specs=pl.BlockSpec((TM, DOUT), lambda t, eot: (t, 0)),
    )
    return pl.pallas_call(
        _ffn_body,
        grid_spec=grid_spec,
        out_shape=jax.ShapeDtypeStruct((P, DOUT), jnp.float32),
    )(eot, xs, w1, b1.reshape(E, 1, DFF), w2, b2.reshape(E, 1, DOUT))


def _sc_scatter(rows, idx, nslots, d):
    n = rows.shape[0]
    info = plsc.get_sparse_core_info()
    nw = info.num_cores * info.num_subcores
    bpw = n // nw
    mesh = plsc.VectorSubcoreMesh(core_axis_name="c", subcore_axis_name="s")

    @functools.partial(
        pl.kernel,
        mesh=mesh,
        out_type=jax.ShapeDtypeStruct((nslots, d), jnp.float32),
        scratch_types=[
            pltpu.VMEM((bpw,), jnp.int32),
            pltpu.VMEM((bpw, d), jnp.float32),
            pltpu.SemaphoreType.DMA,
        ],
    )
    def k(rows_hbm, idx_hbm, out_hbm, idx_v, rows_v, sem):
        wid = lax.axis_index("s") * info.num_cores + lax.axis_index("c")
        base = wid * bpw
        pltpu.sync_copy(idx_hbm.at[pl.ds(base, bpw)], idx_v)
        pltpu.sync_copy(rows_hbm.at[pl.ds(base, bpw)], rows_v)
        pltpu.async_copy(rows_v, out_hbm.at[idx_v], sem).wait()

    return k(rows, idx)


def _sc_gather(table, idx, nrows, d):
    info = plsc.get_sparse_core_info()
    nw = info.num_cores * info.num_subcores
    bpw = nrows // nw
    mesh = plsc.VectorSubcoreMesh(core_axis_name="c", subcore_axis_name="s")

    @functools.partial(
        pl.kernel,
        mesh=mesh,
        out_type=jax.ShapeDtypeStruct((nrows, d), jnp.float32),
        scratch_types=[
            pltpu.VMEM((bpw,), jnp.int32),
            pltpu.VMEM((bpw, d), jnp.float32),
            pltpu.SemaphoreType.DMA,
        ],
    )
    def k(table_hbm, idx_hbm, out_hbm, idx_v, rows_v, sem):
        wid = lax.axis_index("s") * info.num_cores + lax.axis_index("c")
        base = wid * bpw
        pltpu.sync_copy(idx_hbm.at[pl.ds(base, bpw)], idx_v)
        pltpu.async_copy(table_hbm.at[idx_v], rows_v, sem).wait()
        pltpu.sync_copy(rows_v, out_hbm.at[pl.ds(base, bpw)])

    return k(table, idx)


def kernel(x, Wg, bg, W1, b1, W2, b2):
    x2d = x.reshape(S, DIN)
    wgp = jnp.pad(Wg, ((0, 0), (0, EP - E)))
    bgp = jnp.pad(bg, (0, EP - E), constant_values=-1e30).reshape(1, EP)

    dest, cnt = _route(x2d, wgp, bgp)

    c8 = cnt[0, :E]
    pc = ((c8 + TM - 1) // TM) * TM
    ends = jnp.cumsum(pc)
    tile_start = jnp.arange(NT, dtype=jnp.int32) * TM
    eot = jnp.clip(jnp.searchsorted(ends, tile_start, side="right"),
                   0, E - 1).astype(jnp.int32)

    xs = _sc_scatter(x2d, dest.reshape(S), P, DIN)
    ys = _ffn(eot, xs, W1, b1, W2, b2)
    out2d = _sc_gather(ys, dest.reshape(S), S, DOUT)
    return out2d.reshape(1, S, DOUT)


# speedup vs baseline: 1.7840x; 1.0099x over previous
import functools

import jax
import jax.numpy as jnp
from jax import lax
from jax.experimental import pallas as pl
from jax.experimental.pallas import tpu as pltpu
from jax.experimental.pallas import tpu_sc as plsc

E = 8
DIN = 768
DFF = 3072
DOUT = 768
S = 2048
TM = 128
EP = 128
P = S + E * TM
NT = P // TM
RC = 256
NCHUNK = S // RC


def _route_body(x_ref, wg_ref, bg_ref, dest_ref, cnt_ref, oh_ref):
    logits = jnp.dot(x_ref[...], wg_ref[...],
                     preferred_element_type=jnp.float32) + bg_ref[...]
    lane = lax.broadcasted_iota(jnp.int32, (S, EP), 1)
    rowmax = jnp.max(logits, axis=1, keepdims=True)
    ismax = logits >= rowmax
    sel = jnp.min(jnp.where(ismax, lane, EP), axis=1, keepdims=True)
    oh = (lane == sel).astype(jnp.float32)
    oh_ref[...] = oh
    counts = jnp.sum(oh, axis=0, keepdims=True)
    padded = jnp.ceil(counts / TM) * TM
    r2 = lax.broadcasted_iota(jnp.int32, (EP, EP), 0)
    c2 = lax.broadcasted_iota(jnp.int32, (EP, EP), 1)
    starts = jnp.dot(padded.astype(jnp.bfloat16),
                     (r2 < c2).astype(jnp.bfloat16),
                     preferred_element_type=jnp.float32)
    rr = lax.broadcasted_iota(jnp.int32, (RC, RC), 0)
    rc = lax.broadcasted_iota(jnp.int32, (RC, RC), 1)
    ltri = (rr > rc).astype(jnp.bfloat16)

    def body(i, running):
        oh_c = oh_ref[pl.ds(i * RC, RC), :]
        rank_c = jnp.dot(ltri, oh_c.astype(jnp.bfloat16),
                         preferred_element_type=jnp.float32) + running
        dest_c = jnp.sum(oh_c * (rank_c + starts), axis=1, keepdims=True)
        dest_ref[pl.ds(i * RC, RC), :] = dest_c.astype(jnp.int32)
        return running + jnp.sum(oh_c, axis=0, keepdims=True)

    lax.fori_loop(0, NCHUNK, body, jnp.zeros((1, EP), jnp.float32))
    cnt_ref[...] = counts.astype(jnp.int32)


def _route(x2d, wgp, bgp):
    return pl.pallas_call(
        _route_body,
        out_shape=(
            jax.ShapeDtypeStruct((S, 1), jnp.int32),
            jax.ShapeDtypeStruct((1, EP), jnp.int32),
        ),
        scratch_shapes=[pltpu.VMEM((S, EP), jnp.float32)],
    )(x2d, wgp, bgp)


def _ffn_body(eot_ref, xs_ref, w1_ref, b1_ref, w2_ref, b2_ref, out_ref):
    del eot_ref
    a = xs_ref[...].astype(jnp.bfloat16)
    h = jnp.dot(a, w1_ref[0].astype(jnp.bfloat16),
                preferred_element_type=jnp.float32) + b1_ref[0]
    g = 0.5 * h * (1.0 + lax.erf(h * (2.0 ** -0.5)))
    y = jnp.dot(g.astype(jnp.bfloat16), w2_ref[0].astype(jnp.bfloat16),
                preferred_element_type=jnp.float32) + b2_ref[0]
    out_ref[...] = y


def _ffn(eot, xs, w1, b1, w2, b2):
    grid_spec = pltpu.PrefetchScalarGridSpec(
        num_scalar_prefetch=1,
        grid=(NT,),
        in_specs=[
            pl.BlockSpec((TM, DIN), lambda t, eot: (t, 0)),
            pl.BlockSpec((1, DIN, DFF), lambda t, eot: (eot[t], 0, 0)),
            pl.BlockSpec((1, 1, DFF), lambda t, eot: (eot[t], 0, 0)),
            pl.BlockSpec((1, DFF, DOUT), lambda t, eot: (eot[t], 0, 0)),
            pl.BlockSpec((1, 1, DOUT), lambda t, eot: (eot[t], 0, 0)),
        ],
        out_specs=pl.BlockSpec((TM, DOUT), lambda t, eot: (t, 0)),
    )
    return pl.pallas_call(
        _ffn_body,
        grid_spec=grid_spec,
        out_shape=jax.ShapeDtypeStruct((P, DOUT), jnp.float32),
    )(eot, xs, w1, b1.reshape(E, 1, DFF), w2, b2.reshape(E, 1, DOUT))


def _sc_scatter(rows, idx, nslots, d):
    n = rows.shape[0]
    info = plsc.get_sparse_core_info()
    nw = info.num_cores * info.num_subcores
    bpw = n // nw
    mesh = plsc.VectorSubcoreMesh(core_axis_name="c", subcore_axis_name="s")

    @functools.partial(
        pl.kernel,
        mesh=mesh,
        out_type=jax.ShapeDtypeStruct((nslots, d), jnp.float32),
        scratch_types=[
            pltpu.VMEM((bpw,), jnp.int32),
            pltpu.VMEM((bpw, d), jnp.float32),
            pltpu.SemaphoreType.DMA,
        ],
    )
    def k(rows_hbm, idx_hbm, out_hbm, idx_v, rows_v, sem):
        wid = lax.axis_index("s") * info.num_cores + lax.axis_index("c")
        base = wid * bpw
        pltpu.sync_copy(idx_hbm.at[pl.ds(base, bpw)], idx_v)
        pltpu.sync_copy(rows_hbm.at[pl.ds(base, bpw)], rows_v)
        pltpu.async_copy(rows_v, out_hbm.at[idx_v], sem).wait()

    return k(rows, idx)


def _sc_gather(table, idx, nrows, d):
    info = plsc.get_sparse_core_info()
    nw = info.num_cores * info.num_subcores
    bpw = nrows // nw
    mesh = plsc.VectorSubcoreMesh(core_axis_name="c", subcore_axis_name="s")

    @functools.partial(
        pl.kernel,
        mesh=mesh,
        out_type=jax.ShapeDtypeStruct((nrows, d), jnp.float32),
        scratch_types=[
            pltpu.VMEM((bpw,), jnp.int32),
            pltpu.VMEM((bpw, d), jnp.float32),
            pltpu.SemaphoreType.DMA,
        ],
    )
    def k(table_hbm, idx_hbm, out_hbm, idx_v, rows_v, sem):
        wid = lax.axis_index("s") * info.num_cores + lax.axis_index("c")
        base = wid * bpw
        pltpu.sync_copy(idx_hbm.at[pl.ds(base, bpw)], idx_v)
        pltpu.async_copy(table_hbm.at[idx_v], rows_v, sem).wait()
        pltpu.sync_copy(rows_v, out_hbm.at[pl.ds(base, bpw)])

    return k(table, idx)


def kernel(x, Wg, bg, W1, b1, W2, b2):
    x2d = x.reshape(S, DIN)
    wgp = jnp.pad(Wg, ((0, 0), (0, EP - E)))
    bgp = jnp.pad(bg, (0, EP - E), constant_values=-1e30).reshape(1, EP)

    dest, cnt = _route(x2d, wgp, bgp)

    c8 = cnt[0, :E]
    pc = ((c8 + TM - 1) // TM) * TM
    ends = jnp.cumsum(pc)
    tile_start = jnp.arange(NT, dtype=jnp.int32) * TM
    eot = jnp.clip(jnp.searchsorted(ends, tile_start, side="right"),
                   0, E - 1).astype(jnp.int32)

    xs = _sc_scatter(x2d, dest.reshape(S), P, DIN)
    ys = _ffn(eot, xs, W1, b1, W2, b2)
    out2d = _sc_gather(ys, dest.reshape(S), S, DOUT)
    return out2d.reshape(1, S, DOUT)


# speedup vs baseline: 1.8018x; 1.0100x over previous
import functools

import jax
import jax.numpy as jnp
from jax import lax
from jax.experimental import pallas as pl
from jax.experimental.pallas import tpu as pltpu
from jax.experimental.pallas import tpu_sc as plsc

E = 8
DIN = 768
DFF = 3072
DOUT = 768
S = 2048
TM = 128
EP = 128
P = S + E * TM
NT = P // TM
RC = 512
NCHUNK = S // RC


def _route_body(x_ref, wg_ref, bg_ref, dest_ref, cnt_ref, oh_ref):
    logits = jnp.dot(x_ref[...], wg_ref[...],
                     preferred_element_type=jnp.float32) + bg_ref[...]
    lane = lax.broadcasted_iota(jnp.int32, (S, EP), 1)
    rowmax = jnp.max(logits, axis=1, keepdims=True)
    ismax = logits >= rowmax
    sel = jnp.min(jnp.where(ismax, lane, EP), axis=1, keepdims=True)
    oh = (lane == sel).astype(jnp.float32)
    oh_ref[...] = oh
    counts = jnp.sum(oh, axis=0, keepdims=True)
    padded = jnp.ceil(counts / TM) * TM
    r2 = lax.broadcasted_iota(jnp.int32, (EP, EP), 0)
    c2 = lax.broadcasted_iota(jnp.int32, (EP, EP), 1)
    starts = jnp.dot(padded.astype(jnp.bfloat16),
                     (r2 < c2).astype(jnp.bfloat16),
                     preferred_element_type=jnp.float32)
    rr = lax.broadcasted_iota(jnp.int32, (RC, RC), 0)
    rc = lax.broadcasted_iota(jnp.int32, (RC, RC), 1)
    ltri = (rr > rc).astype(jnp.bfloat16)

    def body(i, running):
        oh_c = oh_ref[pl.ds(i * RC, RC), :]
        rank_c = jnp.dot(ltri, oh_c.astype(jnp.bfloat16),
                         preferred_element_type=jnp.float32) + running
        dest_c = jnp.sum(oh_c * (rank_c + starts), axis=1, keepdims=True)
        dest_ref[pl.ds(i * RC, RC), :] = dest_c.astype(jnp.int32)
        return running + jnp.sum(oh_c, axis=0, keepdims=True)

    lax.fori_loop(0, NCHUNK, body, jnp.zeros((1, EP), jnp.float32))
    cnt_ref[...] = counts.astype(jnp.int32)


def _route(x2d, wgp, bgp):
    return pl.pallas_call(
        _route_body,
        out_shape=(
            jax.ShapeDtypeStruct((S, 1), jnp.int32),
            jax.ShapeDtypeStruct((1, EP), jnp.int32),
        ),
        scratch_shapes=[pltpu.VMEM((S, EP), jnp.float32)],
    )(x2d, wgp, bgp)


def _ffn_body(eot_ref, xs_ref, w1_ref, b1_ref, w2_ref, b2_ref, out_ref):
    del eot_ref
    a = xs_ref[...].astype(jnp.bfloat16)
    h = jnp.dot(a, w1_ref[0].astype(jnp.bfloat16),
                preferred_element_type=jnp.float32) + b1_ref[0]
    g = 0.5 * h * (1.0 + lax.erf(h * (2.0 ** -0.5)))
    y = jnp.dot(g.astype(jnp.bfloat16), w2_ref[0].astype(jnp.bfloat16),
                preferred_element_type=jnp.float32) + b2_ref[0]
    out_ref[...] = y


def _ffn(eot, xs, w1, b1, w2, b2):
    grid_spec = pltpu.PrefetchScalarGridSpec(
        num_scalar_prefetch=1,
        grid=(NT,),
        in_specs=[
            pl.BlockSpec((TM, DIN), lambda t, eot: (t, 0)),
            pl.BlockSpec((1, DIN, DFF), lambda t, eot: (eot[t], 0, 0)),
            pl.BlockSpec((1, 1, DFF), lambda t, eot: (eot[t], 0, 0)),
            pl.BlockSpec((1, DFF, DOUT), lambda t, eot: (eot[t], 0, 0)),
            pl.BlockSpec((1, 1, DOUT), lambda t, eot: (eot[t], 0, 0)),
        ],
        out_specs=pl.BlockSpec((TM, DOUT), lambda t, eot: (t, 0)),
    )
    return pl.pallas_call(
        _ffn_body,
        grid_spec=grid_spec,
        out_shape=jax.ShapeDtypeStruct((P, DOUT), jnp.float32),
    )(eot, xs, w1, b1.reshape(E, 1, DFF), w2, b2.reshape(E, 1, DOUT))


def _sc_scatter(rows, idx, nslots, d):
    n = rows.shape[0]
    info = plsc.get_sparse_core_info()
    nw = info.num_cores * info.num_subcores
    bpw = n // nw
    mesh = plsc.VectorSubcoreMesh(core_axis_name="c", subcore_axis_name="s")

    @functools.partial(
        pl.kernel,
        mesh=mesh,
        out_type=jax.ShapeDtypeStruct((nslots, d), jnp.float32),
        scratch_types=[
            pltpu.VMEM((bpw,), jnp.int32),
            pltpu.VMEM((bpw, d), jnp.float32),
            pltpu.SemaphoreType.DMA,
        ],
    )
    def k(rows_hbm, idx_hbm, out_hbm, idx_v, rows_v, sem):
        wid = lax.axis_index("s") * info.num_cores + lax.axis_index("c")
        base = wid * bpw
        pltpu.sync_copy(idx_hbm.at[pl.ds(base, bpw)], idx_v)
        pltpu.sync_copy(rows_hbm.at[pl.ds(base, bpw)], rows_v)
        pltpu.async_copy(rows_v, out_hbm.at[idx_v], sem).wait()

    return k(rows, idx)


def _sc_gather(table, idx, nrows, d):
    info = plsc.get_sparse_core_info()
    nw = info.num_cores * info.num_subcores
    bpw = nrows // nw
    mesh = plsc.VectorSubcoreMesh(core_axis_name="c", subcore_axis_name="s")

    @functools.partial(
        pl.kernel,
        mesh=mesh,
        out_type=jax.ShapeDtypeStruct((nrows, d), jnp.float32),
        scratch_types=[
            pltpu.VMEM((bpw,), jnp.int32),
            pltpu.VMEM((bpw, d), jnp.float32),
            pltpu.SemaphoreType.DMA,
        ],
    )
    def k(table_hbm, idx_hbm, out_hbm, idx_v, rows_v, sem):
        wid = lax.axis_index("s") * info.num_cores + lax.axis_index("c")
        base = wid * bpw
        pltpu.sync_copy(idx_hbm.at[pl.ds(base, bpw)], idx_v)
        pltpu.async_copy(table_hbm.at[idx_v], rows_v, sem).wait()
        pltpu.sync_copy(rows_v, out_hbm.at[pl.ds(base, bpw)])

    return k(table, idx)


def kernel(x, Wg, bg, W1, b1, W2, b2):
    x2d = x.reshape(S, DIN)
    wgp = jnp.pad(Wg, ((0, 0), (0, EP - E)))
    bgp = jnp.pad(bg, (0, EP - E), constant_values=-1e30).reshape(1, EP)

    dest, cnt = _route(x2d, wgp, bgp)

    c8 = cnt[0, :E]
    pc = ((c8 + TM - 1) // TM) * TM
    ends = jnp.cumsum(pc)
    tile_start = jnp.arange(NT, dtype=jnp.int32) * TM
    eot = jnp.clip(jnp.searchsorted(ends, tile_start, side="right"),
                   0, E - 1).astype(jnp.int32)

    xs = _sc_scatter(x2d, dest.reshape(S), P, DIN)
    ys = _ffn(eot, xs, W1, b1, W2, b2)
    out2d = _sc_gather(ys, dest.reshape(S), S, DOUT)
    return out2d.reshape(1, S, DOUT)
